# baseline (speedup 1.0000x reference)
import functools

import jax
import jax.numpy as jnp
import numpy as np
from jax import lax
from jax.experimental import pallas as pl
from jax.experimental.pallas import tpu as pltpu
from jax.experimental.pallas import tpu_sc as plsc

N_NODES = 10000
N_EDGES = 320000
D_FEAT = 128
D_EDGE = 16
D_OUT = 128
_DP = D_OUT // 2

_NC = 2
_NS = 16
_NW = _NC * _NS

_C = 128
_NCH = N_EDGES // _C
_NK = 80
_L = 16


def _tc_body(x_ref, w1_ref, w2_ref, ea_ref, w3l_ref, w3h_ref, bl_ref,
             bh_ref, t12_ref, e3_ref):
    @pl.when(pl.program_id(0) == 0)
    def _():
        xv = x_ref[...]
        t12_ref[pl.ds(0, N_NODES), :] = jnp.dot(
            xv, w1_ref[...], preferred_element_type=jnp.float32)
        t12_ref[pl.ds(N_NODES, N_NODES), :] = jnp.dot(
            xv, w2_ref[...], preferred_element_type=jnp.float32)

    ea = ea_ref[...]
    lo = (jnp.dot(ea, w3l_ref[...], preferred_element_type=jnp.float32)
          + bl_ref[...]).astype(jnp.bfloat16)
    hi = (jnp.dot(ea, w3h_ref[...], preferred_element_type=jnp.float32)
          + bh_ref[...]).astype(jnp.bfloat16)
    lo_i = lax.bitcast_convert_type(lo, jnp.int16).astype(jnp.int32) & 0xFFFF
    hi_i = lax.bitcast_convert_type(hi, jnp.int16).astype(jnp.int32) << 16
    e3_ref[...] = lo_i | hi_i


_EB = 8000

_LOCOLS = np.concatenate(
    [np.arange(m * 32, m * 32 + 16) for m in range(4)]).astype(np.int32)
_HICOLS = _LOCOLS + 16

_CIDS = np.minimum(
    np.arange(_NW)[:, None] + _NW * np.arange(_NK)[None, :], _NCH - 1)


def _sc_body(t_hbm, e3_hbm, srw_hbm, out_hbm,
             idxall, g12, acc,
             sem_i, sem_g0, sem_g1, sem_w0, sem_w1):
    wid = lax.axis_index("s") * _NC + lax.axis_index("c")
    sem_g = (sem_g0, sem_g1)
    sem_w = (sem_w0, sem_w1)

    def chunk_base(k):
        return jnp.minimum(wid + k * _NW, _NCH - 1) * _C

    def issue_g(k, b):
        base = chunk_base(k)
        idx = idxall.at[k]
        pltpu.async_copy(e3_hbm.at[pl.ds(base, _C)], acc.at[b], sem_g[b])
        pltpu.async_copy(t_hbm.at[idx.at[0]], g12.at[b].at[0], sem_g[b])
        pltpu.async_copy(t_hbm.at[idx.at[1]], g12.at[b].at[1], sem_g[b])

    def wait_g(b):
        idx = idxall.at[0]
        pltpu.make_async_copy(e3_hbm.at[pl.ds(0, _C)], acc.at[b], sem_g[b]).wait()
        pltpu.make_async_copy(
            t_hbm.at[idx.at[0]], g12.at[b].at[0], sem_g[b]).wait()
        pltpu.make_async_copy(
            t_hbm.at[idx.at[1]], g12.at[b].at[1], sem_g[b]).wait()

    def issue_wb(k, b):
        base = chunk_base(k)
        pltpu.async_copy(
            g12.at[b].at[0], out_hbm.at[pl.ds(base, _C)], sem_w[b])

    def wait_wb(b):
        pltpu.make_async_copy(
            g12.at[b].at[0], out_hbm.at[pl.ds(0, _C)], sem_w[b]).wait()

    def compute(b):
        accb, g1b, g2b = acc.at[b], g12.at[b].at[0], g12.at[b].at[1]
        msk = jnp.full((_L,), -65536, dtype=jnp.int32)
        sh = jnp.full((_L,), 16, dtype=jnp.int32)

        def up_lo(v):
            return lax.bitcast_convert_type(lax.shift_left(v, sh), jnp.float32)

        def up_hi(v):
            return lax.bitcast_convert_type(
                lax.bitwise_and(v, msk), jnp.float32)

        @pl.loop(0, _C, unroll=4)
        def _(i):
            for m in range(D_OUT // 32):
                ve = accb[i, pl.ds(_L * m, _L)]
                lo = pl.ds(32 * m, _L)
                hi = pl.ds(32 * m + _L, _L)
                g1b[i, lo] = jnp.maximum(
                    (g1b[i, lo] + g2b[i, lo]) + up_lo(ve), 0.0)
                g1b[i, hi] = jnp.maximum(
                    (g1b[i, hi] + g2b[i, hi]) + up_hi(ve), 0.0)

    pltpu.async_copy(srw_hbm.at[wid], idxall, sem_i).wait()
    issue_g(0, 0)

    @pl.loop(0, _NK, step=2)
    def _(kk):
        for d in range(2):
            k = kk + d
            b = d
            bn = 1 - d

            wait_g(b)

            @pl.when(k + 1 < _NK)
            def _():
                @pl.when(k >= 1)
                def _():
                    wait_wb(bn)

                issue_g(k + 1, bn)

            compute(b)
            issue_wb(k, b)

    wait_wb(0)
    wait_wb(1)


@jax.jit
def _impl(x, s_idx, r_idx, edge_attr, W, b):
    w1 = W[0:D_FEAT]
    w2 = W[D_FEAT:2 * D_FEAT]
    w3 = W[2 * D_FEAT:]

    t12, e3 = pl.pallas_call(
        _tc_body,
        grid=(N_EDGES // _EB,),
        in_specs=[
            pl.BlockSpec((N_NODES, D_FEAT), lambda i: (0, 0)),
            pl.BlockSpec((D_FEAT, D_FEAT), lambda i: (0, 0)),
            pl.BlockSpec((D_FEAT, D_FEAT), lambda i: (0, 0)),
            pl.BlockSpec((_EB, D_EDGE), lambda i: (i, 0)),
            pl.BlockSpec((D_EDGE, _DP), lambda i: (0, 0)),
            pl.BlockSpec((D_EDGE, _DP), lambda i: (0, 0)),
            pl.BlockSpec((1, _DP), lambda i: (0, 0)),
            pl.BlockSpec((1, _DP), lambda i: (0, 0)),
        ],
        out_specs=(
            pl.BlockSpec((2 * N_NODES, D_FEAT), lambda i: (0, 0)),
            pl.BlockSpec((_EB, _DP), lambda i: (i, 0)),
        ),
        out_shape=(
            jax.ShapeDtypeStruct((2 * N_NODES, D_FEAT), jnp.float32),
            jax.ShapeDtypeStruct((N_EDGES, _DP), jnp.int32),
        ),
    )(
        x, w1, w2,
        edge_attr,
        w3[:, _LOCOLS],
        w3[:, _HICOLS],
        b[_LOCOLS].reshape(1, _DP),
        b[_HICOLS].reshape(1, _DP),
    )

    sc_fuse = functools.partial(
        pl.kernel,
        out_type=jax.ShapeDtypeStruct((N_EDGES, D_OUT), jnp.float32),
        mesh=plsc.VectorSubcoreMesh(
            core_axis_name="c", subcore_axis_name="s",
            num_cores=_NC, num_subcores=_NS,
        ),
        scratch_types=[
            pltpu.VMEM((_NK, 2, _C), jnp.int32),
            pltpu.VMEM((2, 2, _C, D_OUT), jnp.float32),
            pltpu.VMEM((2, _C, _DP), jnp.int32),
            pltpu.SemaphoreType.DMA,
            pltpu.SemaphoreType.DMA,
            pltpu.SemaphoreType.DMA,
            pltpu.SemaphoreType.DMA,
            pltpu.SemaphoreType.DMA,
        ],
    )(_sc_body)

    sr = jnp.stack(
        [s_idx.reshape(_NCH, _C), r_idx.reshape(_NCH, _C) + N_NODES], axis=1)
    pad = jnp.broadcast_to(sr[-1], (_NK * _NW - _NCH, 2, _C))
    srw = jnp.concatenate([sr, pad], axis=0).reshape(
        _NK, _NW, 2, _C).transpose(1, 0, 2, 3)
    return sc_fuse(t12, e3, srw)


def kernel(x, edge_index, edge_attr, W, b):
    s_idx = edge_index[0].astype(jnp.int32)
    r_idx = edge_index[1].astype(jnp.int32)
    return _impl(x, s_idx, r_idx, edge_attr, W, b)


# speedup vs baseline: 1.0276x; 1.0276x over previous
import functools

import jax
import jax.numpy as jnp
import numpy as np
from jax import lax
from jax.experimental import pallas as pl
from jax.experimental.pallas import tpu as pltpu
from jax.experimental.pallas import tpu_sc as plsc

N_NODES = 10000
N_EDGES = 320000
D_FEAT = 128
D_EDGE = 16
D_OUT = 128
_DP = D_OUT // 2

_NC = 2
_NS = 16
_NW = _NC * _NS

_C = 128
_NCH = N_EDGES // _C
_NK = 80
_L = 16


def _tc_body(x_ref, w1_ref, w2_ref, ea_ref, w3l_ref, w3h_ref, bl_ref,
             bh_ref, t12_ref, e3_ref):
    @pl.when(pl.program_id(0) == 0)
    def _():
        xv = x_ref[...]
        t12_ref[pl.ds(0, N_NODES), :] = jnp.dot(
            xv, w1_ref[...], preferred_element_type=jnp.float32)
        t12_ref[pl.ds(N_NODES, N_NODES), :] = jnp.dot(
            xv, w2_ref[...], preferred_element_type=jnp.float32)

    ea = ea_ref[...]
    lo = (jnp.dot(ea, w3l_ref[...], preferred_element_type=jnp.float32)
          + bl_ref[...]).astype(jnp.bfloat16)
    hi = (jnp.dot(ea, w3h_ref[...], preferred_element_type=jnp.float32)
          + bh_ref[...]).astype(jnp.bfloat16)
    lo_i = lax.bitcast_convert_type(lo, jnp.int16).astype(jnp.int32) & 0xFFFF
    hi_i = lax.bitcast_convert_type(hi, jnp.int16).astype(jnp.int32) << 16
    e3_ref[...] = lo_i | hi_i


_EB = 8000

_LOCOLS = np.concatenate(
    [np.arange(m * 32, m * 32 + 16) for m in range(4)]).astype(np.int32)
_HICOLS = _LOCOLS + 16


def _sc_body(t_hbm, e3_hbm, sr_hbm, out_hbm,
             ivc, g12, acc,
             sem_i0, sem_i1, sem_g0, sem_g1, sem_w0, sem_w1):
    wid = lax.axis_index("s") * _NC + lax.axis_index("c")
    sem_i = (sem_i0, sem_i1)
    sem_g = (sem_g0, sem_g1)
    sem_w = (sem_w0, sem_w1)

    def chunk_base(k):
        return jnp.minimum(wid + k * _NW, _NCH - 1) * _C

    def chunk_id(k):
        return jnp.minimum(wid + k * _NW, _NCH - 1)

    def issue_idx(k, b):
        pltpu.async_copy(sr_hbm.at[chunk_id(k)], ivc.at[b], sem_i[b])

    def wait_idx(b):
        pltpu.make_async_copy(sr_hbm.at[0], ivc.at[b], sem_i[b]).wait()

    def issue_g(k, b):
        base = chunk_base(k)
        pltpu.async_copy(e3_hbm.at[pl.ds(base, _C)], acc.at[b], sem_g[b])
        pltpu.async_copy(t_hbm.at[ivc.at[b].at[0]], g12.at[b].at[0], sem_g[b])
        pltpu.async_copy(t_hbm.at[ivc.at[b].at[1]], g12.at[b].at[1], sem_g[b])

    def wait_g(b):
        pltpu.make_async_copy(e3_hbm.at[pl.ds(0, _C)], acc.at[b], sem_g[b]).wait()
        pltpu.make_async_copy(
            t_hbm.at[ivc.at[b].at[0]], g12.at[b].at[0], sem_g[b]).wait()
        pltpu.make_async_copy(
            t_hbm.at[ivc.at[b].at[1]], g12.at[b].at[1], sem_g[b]).wait()

    def issue_wb(k, b):
        base = chunk_base(k)
        pltpu.async_copy(
            g12.at[b].at[0], out_hbm.at[pl.ds(base, _C)], sem_w[b])

    def wait_wb(b):
        pltpu.make_async_copy(
            g12.at[b].at[0], out_hbm.at[pl.ds(0, _C)], sem_w[b]).wait()

    def compute(b):
        accb, g1b, g2b = acc.at[b], g12.at[b].at[0], g12.at[b].at[1]
        msk = jnp.full((_L,), -65536, dtype=jnp.int32)
        sh = jnp.full((_L,), 16, dtype=jnp.int32)

        def up_lo(v):
            return lax.bitcast_convert_type(lax.shift_left(v, sh), jnp.float32)

        def up_hi(v):
            return lax.bitcast_convert_type(
                lax.bitwise_and(v, msk), jnp.float32)

        @pl.loop(0, _C, unroll=4)
        def _(i):
            for m in range(D_OUT // 32):
                ve = accb[i, pl.ds(_L * m, _L)]
                lo = pl.ds(32 * m, _L)
                hi = pl.ds(32 * m + _L, _L)
                g1b[i, lo] = jnp.maximum(
                    (g1b[i, lo] + g2b[i, lo]) + up_lo(ve), 0.0)
                g1b[i, hi] = jnp.maximum(
                    (g1b[i, hi] + g2b[i, hi]) + up_hi(ve), 0.0)

    issue_idx(0, 0)
    issue_idx(1, 1)
    wait_idx(0)
    issue_g(0, 0)

    @pl.loop(0, _NK, step=2)
    def _(kk):
        for d in range(2):
            k = kk + d
            b = d
            bn = 1 - d

            wait_g(b)

            @pl.when(k + 1 < _NK)
            def _():
                wait_idx(bn)

                @pl.when(k >= 1)
                def _():
                    wait_wb(bn)

                issue_g(k + 1, bn)

            @pl.when(k + 2 < _NK)
            def _():
                issue_idx(k + 2, b)

            compute(b)
            issue_wb(k, b)

    wait_wb(0)
    wait_wb(1)


@jax.jit
def _impl(x, s_idx, r_idx, edge_attr, W, b):
    w1 = W[0:D_FEAT]
    w2 = W[D_FEAT:2 * D_FEAT]
    w3 = W[2 * D_FEAT:]

    t12, e3 = pl.pallas_call(
        _tc_body,
        grid=(N_EDGES // _EB,),
        in_specs=[
            pl.BlockSpec((N_NODES, D_FEAT), lambda i: (0, 0)),
            pl.BlockSpec((D_FEAT, D_FEAT), lambda i: (0, 0)),
            pl.BlockSpec((D_FEAT, D_FEAT), lambda i: (0, 0)),
            pl.BlockSpec((_EB, D_EDGE), lambda i: (i, 0)),
            pl.BlockSpec((D_EDGE, _DP), lambda i: (0, 0)),
            pl.BlockSpec((D_EDGE, _DP), lambda i: (0, 0)),
            pl.BlockSpec((1, _DP), lambda i: (0, 0)),
            pl.BlockSpec((1, _DP), lambda i: (0, 0)),
        ],
        out_specs=(
            pl.BlockSpec((2 * N_NODES, D_FEAT), lambda i: (0, 0)),
            pl.BlockSpec((_EB, _DP), lambda i: (i, 0)),
        ),
        out_shape=(
            jax.ShapeDtypeStruct((2 * N_NODES, D_FEAT), jnp.float32),
            jax.ShapeDtypeStruct((N_EDGES, _DP), jnp.int32),
        ),
    )(
        x, w1, w2,
        edge_attr,
        w3[:, _LOCOLS],
        w3[:, _HICOLS],
        b[_LOCOLS].reshape(1, _DP),
        b[_HICOLS].reshape(1, _DP),
    )

    sc_fuse = functools.partial(
        pl.kernel,
        out_type=jax.ShapeDtypeStruct((N_EDGES, D_OUT), jnp.float32),
        mesh=plsc.VectorSubcoreMesh(
            core_axis_name="c", subcore_axis_name="s",
            num_cores=_NC, num_subcores=_NS,
        ),
        scratch_types=[
            pltpu.VMEM((2, 2, _C), jnp.int32),
            pltpu.VMEM((2, 2, _C, D_OUT), jnp.float32),
            pltpu.VMEM((2, _C, _DP), jnp.int32),
            pltpu.SemaphoreType.DMA,
            pltpu.SemaphoreType.DMA,
            pltpu.SemaphoreType.DMA,
            pltpu.SemaphoreType.DMA,
            pltpu.SemaphoreType.DMA,
            pltpu.SemaphoreType.DMA,
        ],
    )(_sc_body)

    sr = jnp.stack(
        [s_idx.reshape(_NCH, _C), r_idx.reshape(_NCH, _C) + N_NODES], axis=1)
    return sc_fuse(t12, e3, sr)


def kernel(x, edge_index, edge_attr, W, b):
    s_idx = edge_index[0].astype(jnp.int32)
    r_idx = edge_index[1].astype(jnp.int32)
    return _impl(x, s_idx, r_idx, edge_attr, W, b)
